# baseline (speedup 1.0000x reference)
import functools

import jax
import jax.numpy as jnp
from jax import lax
from jax.experimental import pallas as pl
from jax.experimental.pallas import tpu as pltpu
from jax.experimental.pallas import tpu_sc as plsc

N = 10000
E = 320000
D_IN = 128
D_HID = 256
D_OUT = 128

NC = 2
NS = 16
K = 128
E_T = E // NS
STEPS = 160
PAD = STEPS * K - E_T
N_ACC = 10240
ROWS_PER_TILE = N_ACC // NS
NB = 10
BLK = N // NB

_mesh = plsc.VectorSubcoreMesh(
    core_axis_name="c", subcore_axis_name="s", num_cores=NC, num_subcores=NS
)


@functools.partial(
    pl.kernel,
    out_type=jax.ShapeDtypeStruct((N_ACC,), jnp.float32),
    mesh=_mesh,
    scratch_types=[
        pltpu.VMEM((STEPS, K), jnp.int32),
        pltpu.VMEM((K,), jnp.float32),
        pltpu.VMEM((ROWS_PER_TILE,), jnp.float32),
        pltpu.VMEM_SHARED((N_ACC,), jnp.float32),
    ],
)
def _deg_kernel(dst_hbm, deg_out, dst_v, ones_v, zbuf, deg_sp):
    cid = lax.axis_index("c")
    w = lax.axis_index("s")
    zeros16 = jnp.zeros((16,), jnp.float32)
    ones16 = jnp.ones((16,), jnp.float32)

    def zb(i, carry):
        zbuf[pl.ds(i * 16, 16)] = zeros16
        return carry

    lax.fori_loop(0, ROWS_PER_TILE // 16, zb, 0)
    for j in range(K // 16):
        ones_v[pl.ds(j * 16, 16)] = ones16
    pltpu.sync_copy(zbuf, deg_sp.at[pl.ds(w * ROWS_PER_TILE, ROWS_PER_TILE)])
    pltpu.sync_copy(dst_hbm.at[pl.ds(w * STEPS, STEPS)], dst_v)
    plsc.subcore_barrier()

    def step(s, carry):
        pltpu.sync_copy(ones_v, deg_sp.at[dst_v.at[s]], add=True)
        return carry

    lax.fori_loop(0, STEPS, step, 0)
    plsc.subcore_barrier()

    @pl.when(jnp.logical_and(cid == 0, w == 0))
    def _():
        pltpu.sync_copy(deg_sp, deg_out)


N_HALF = 5000
A_ROWS = 5120
A_PT = A_ROWS // NS
DW = 128


def _make_scatter_kernel(n_tables):
    n_out = 4 if n_tables == 2 else 2

    @functools.partial(
        pl.kernel,
        out_type=tuple(
            jax.ShapeDtypeStruct((A_ROWS, DW), jnp.float32)
            for _ in range(n_out)
        ),
        mesh=_mesh,
        scratch_types=[
            pltpu.VMEM((STEPS, K), jnp.int32),
            pltpu.VMEM((STEPS, K), jnp.int32),
            pltpu.VMEM((K,), jnp.int32),
            pltpu.VMEM((K,), jnp.int32),
            pltpu.VMEM((K, DW), jnp.float32),
            pltpu.VMEM((K, DW), jnp.float32),
            pltpu.VMEM((64, DW), jnp.float32),
            pltpu.VMEM_SHARED((A_ROWS, DW), jnp.float32),
            pltpu.SemaphoreType.DMA,
            pltpu.SemaphoreType.DMA,
            pltpu.SemaphoreType.DMA,
            pltpu.SemaphoreType.DMA,
        ],
    )
    def scatter_kernel(*refs):
        ys = refs[:n_tables]
        src_hbm, dst_hbm = refs[n_tables], refs[n_tables + 1]
        outs = refs[n_tables + 2: n_tables + 2 + n_out]
        (src_v, dst_v, widx0, widx1, rows0, rows1, zbuf, acc,
         gsem0, gsem1, ssem0, ssem1) = refs[n_tables + 2 + n_out:]
        cid = lax.axis_index("c")
        w = lax.axis_index("s")
        zeros16 = jnp.zeros((16,), jnp.float32)

        def zb(i, carry):
            for j in range(DW // 16):
                zbuf[i, pl.ds(j * 16, 16)] = zeros16
            return carry

        lax.fori_loop(0, 64, zb, 0)
        pltpu.sync_copy(src_hbm.at[pl.ds(w * STEPS, STEPS)], src_v)
        pltpu.sync_copy(dst_hbm.at[pl.ds(w * STEPS, STEPS)], dst_v)

        def run(y_ref, base, out_ref):
            for i in range(A_PT // 64):
                pltpu.sync_copy(zbuf, acc.at[pl.ds(w * A_PT + i * 64, 64)])
            plsc.subcore_barrier()

            def gather(s, rows_b, sem_b):
                pltpu.async_copy(y_ref.at[src_v.at[s]], rows_b, sem_b)

            def gwait(rows_b, sem_b):
                pltpu.make_async_copy(
                    y_ref.at[src_v.at[0]], rows_b, sem_b
                ).wait()

            def mask(s, widx_b):
                for j in range(K // 16):
                    d16 = dst_v[s, pl.ds(j * 16, 16)]
                    local = d16 - base
                    ok = jnp.logical_and(local >= 0, local < N_HALF)
                    widx_b[pl.ds(j * 16, 16)] = jnp.where(
                        ok, local, N_HALF + (d16 & 63)
                    )

            def scat(rows_b, widx_b, sem_b):
                pltpu.async_copy(rows_b, acc.at[widx_b], sem_b, add=True)

            def swait(rows_b, widx_b, sem_b):
                pltpu.make_async_copy(
                    rows_b, acc.at[widx_b], sem_b
                ).wait()

            gather(0, rows0, gsem0)
            gwait(rows0, gsem0)
            mask(0, widx0)
            scat(rows0, widx0, ssem0)
            gather(1, rows1, gsem1)
            gwait(rows1, gsem1)
            mask(1, widx1)
            scat(rows1, widx1, ssem1)
            swait(rows0, widx0, ssem0)
            gather(2, rows0, gsem0)

            def pair(p, carry):
                s0 = 2 * p
                gwait(rows0, gsem0)
                mask(s0, widx0)
                scat(rows0, widx0, ssem0)
                swait(rows1, widx1, ssem1)
                gather(s0 + 1, rows1, gsem1)
                gwait(rows1, gsem1)
                mask(s0 + 1, widx1)
                scat(rows1, widx1, ssem1)
                swait(rows0, widx0, ssem0)

                @pl.when(p < STEPS // 2 - 1)
                def _():
                    gather(s0 + 2, rows0, gsem0)

                return carry

            lax.fori_loop(1, STEPS // 2, pair, 0)
            swait(rows1, widx1, ssem1)
            plsc.subcore_barrier()
            pltpu.sync_copy(
                acc.at[pl.ds(w * A_PT, A_PT)],
                out_ref.at[pl.ds(w * A_PT, A_PT)],
            )

        if n_tables == 2:
            units = [[(0, 0, 0), (0, N_HALF, 1)], [(1, 0, 2), (1, N_HALF, 3)]]
        else:
            units = [[(0, 0, 0)], [(0, N_HALF, 1)]]
        for c in range(NC):
            @pl.when(cid == c)
            def _(c=c):
                for t, base, o in units[c]:
                    run(ys[t], base, outs[o])

    return scatter_kernel


_scatter_hid = _make_scatter_kernel(2)
_scatter_out = _make_scatter_kernel(1)


def _mm1_body(x_ref, w1_ref, deg_ref, ya_ref, yb_ref):
    dinv = lax.rsqrt(deg_ref[...] + 1.0)
    z = jnp.dot(x_ref[...], w1_ref[...], preferred_element_type=jnp.float32)
    y = z * dinv
    ya_ref[...] = y[:, :DW]
    yb_ref[...] = y[:, DW:]


def _mm2_body(sa_ref, sb_ref, ya_ref, yb_ref, deg_ref, b1_ref, w2_ref,
              y2_ref):
    dinv = lax.rsqrt(deg_ref[...] + 1.0)
    h = jnp.concatenate(
        [sa_ref[...] + ya_ref[...], sb_ref[...] + yb_ref[...]], axis=1
    )
    h = jnp.maximum(h * dinv + b1_ref[...], 0.0)
    y2_ref[...] = (
        jnp.dot(h, w2_ref[...], preferred_element_type=jnp.float32) * dinv
    )


def _final_body(s_ref, y_ref, deg_ref, b2_ref, o_ref):
    dinv = lax.rsqrt(deg_ref[...] + 1.0)
    o_ref[...] = (s_ref[...] + y_ref[...]) * dinv + b2_ref[...]


def _row_spec(d):
    return pl.BlockSpec((BLK, d), lambda i: (i, 0))


def _full_spec(r, c):
    return pl.BlockSpec((r, c), lambda i: (0, 0))


def kernel(x, edge_index, W1, b1, W2, b2):
    src = edge_index[0]
    dst = edge_index[1]
    pad_src = jnp.zeros((NS, PAD), jnp.int32)
    pad_dst = jnp.broadcast_to(
        N + (jnp.arange(PAD, dtype=jnp.int32) % (N_ACC - N)), (NS, PAD)
    )
    src_pad = jnp.concatenate([src.reshape(NS, E_T), pad_src], axis=1)
    src_pad = src_pad.reshape(NS * STEPS, K)
    dst_pad = jnp.concatenate([dst.reshape(NS, E_T), pad_dst], axis=1)
    dst_pad = dst_pad.reshape(NS * STEPS, K)

    deg = _deg_kernel(dst_pad)
    deg_col = deg[:N].reshape(N, 1)

    y1a, y1b = pl.pallas_call(
        _mm1_body,
        grid=(NB,),
        in_specs=[
            _row_spec(D_IN),
            _full_spec(D_IN, D_HID),
            _row_spec(1),
        ],
        out_specs=(_row_spec(DW), _row_spec(DW)),
        out_shape=(
            jax.ShapeDtypeStruct((N, DW), jnp.float32),
            jax.ShapeDtypeStruct((N, DW), jnp.float32),
        ),
    )(x, W1, deg_col)

    u0, u1, u2, u3 = _scatter_hid(y1a, y1b, src_pad, dst_pad)
    s1a = jnp.concatenate([u0[:N_HALF], u1[:N_HALF]], axis=0)
    s1b = jnp.concatenate([u2[:N_HALF], u3[:N_HALF]], axis=0)

    y2 = pl.pallas_call(
        _mm2_body,
        grid=(NB,),
        in_specs=[
            _row_spec(DW),
            _row_spec(DW),
            _row_spec(DW),
            _row_spec(DW),
            _row_spec(1),
            _full_spec(1, D_HID),
            _full_spec(D_HID, D_OUT),
        ],
        out_specs=_row_spec(D_OUT),
        out_shape=jax.ShapeDtypeStruct((N, D_OUT), jnp.float32),
    )(s1a, s1b, y1a, y1b, deg_col, b1.reshape(1, D_HID), W2)

    v0, v1 = _scatter_out(y2, src_pad, dst_pad)
    s2 = jnp.concatenate([v0[:N_HALF], v1[:N_HALF]], axis=0)

    out = pl.pallas_call(
        _final_body,
        grid=(NB,),
        in_specs=[
            _row_spec(D_OUT),
            _row_spec(D_OUT),
            _row_spec(1),
            _full_spec(1, D_OUT),
        ],
        out_specs=_row_spec(D_OUT),
        out_shape=jax.ShapeDtypeStruct((N, D_OUT), jnp.float32),
    )(s2, y2, deg_col, b2.reshape(1, D_OUT))

    return out


# speedup vs baseline: 1.2852x; 1.2852x over previous
import functools

import jax
import jax.numpy as jnp
from jax import lax
from jax.experimental import pallas as pl
from jax.experimental.pallas import tpu as pltpu
from jax.experimental.pallas import tpu_sc as plsc

N = 10000
E = 320000
D_IN = 128
D_HID = 256
D_OUT = 128

NC = 2
NS = 16
K = 128
E_T = E // NS
STEPS = 160
PAD = STEPS * K - E_T
N_ACC = 10240
ROWS_PER_TILE = N_ACC // NS
NB = 10
BLK = N // NB

_mesh = plsc.VectorSubcoreMesh(
    core_axis_name="c", subcore_axis_name="s", num_cores=NC, num_subcores=NS
)


@functools.partial(
    pl.kernel,
    out_type=jax.ShapeDtypeStruct((N_ACC,), jnp.float32),
    mesh=_mesh,
    scratch_types=[
        pltpu.VMEM((STEPS, K), jnp.int32),
        pltpu.VMEM((K,), jnp.float32),
        pltpu.VMEM((ROWS_PER_TILE,), jnp.float32),
        pltpu.VMEM_SHARED((N_ACC,), jnp.float32),
    ],
)
def _deg_kernel(dst_hbm, deg_out, dst_v, ones_v, zbuf, deg_sp):
    cid = lax.axis_index("c")
    w = lax.axis_index("s")
    zeros16 = jnp.zeros((16,), jnp.float32)
    ones16 = jnp.ones((16,), jnp.float32)

    def zb(i, carry):
        zbuf[pl.ds(i * 16, 16)] = zeros16
        return carry

    lax.fori_loop(0, ROWS_PER_TILE // 16, zb, 0)
    for j in range(K // 16):
        ones_v[pl.ds(j * 16, 16)] = ones16
    pltpu.sync_copy(zbuf, deg_sp.at[pl.ds(w * ROWS_PER_TILE, ROWS_PER_TILE)])
    pltpu.sync_copy(dst_hbm.at[pl.ds(w * STEPS, STEPS)], dst_v)
    plsc.subcore_barrier()

    def step(s, carry):
        pltpu.sync_copy(ones_v, deg_sp.at[dst_v.at[s]], add=True)
        return carry

    lax.fori_loop(0, STEPS, step, 0)
    plsc.subcore_barrier()

    @pl.when(jnp.logical_and(cid == 0, w == 0))
    def _():
        pltpu.sync_copy(deg_sp, deg_out)


DW = 128
N_HALF = 5000
A_ROWS = 5120
A_PT = A_ROWS // NS


def _make_scatter_kernel(n_accs, units):
    n_tables = max(u[0] for cu in units for u in cu) + 1
    n_out = n_accs * sum(len(cu) for cu in units)

    @functools.partial(
        pl.kernel,
        out_type=tuple(
            jax.ShapeDtypeStruct((A_ROWS, DW), jnp.float32)
            for _ in range(n_out)
        ),
        mesh=_mesh,
        scratch_types=(
            [
                pltpu.VMEM((STEPS, K), jnp.int32),
                pltpu.VMEM((STEPS, K), jnp.int32),
                pltpu.VMEM((K,), jnp.int32),
                pltpu.VMEM((K,), jnp.int32),
                pltpu.VMEM((K, DW), jnp.float32),
                pltpu.VMEM((K, DW), jnp.float32),
                pltpu.VMEM((64, DW), jnp.float32),
            ]
            + [pltpu.VMEM_SHARED((A_ROWS, DW), jnp.float32)] * n_accs
            + [pltpu.SemaphoreType.DMA, pltpu.SemaphoreType.DMA]
        ),
    )
    def scatter_kernel(*refs):
        ys = refs[:n_tables]
        src_hbm, dst_hbm = refs[n_tables], refs[n_tables + 1]
        outs = refs[n_tables + 2: n_tables + 2 + n_out]
        sc = refs[n_tables + 2 + n_out:]
        src_v, dst_v, widx0, widx1, rows0, rows1, zbuf = sc[:7]
        accs = sc[7: 7 + n_accs]
        gsem0, gsem1 = sc[7 + n_accs:]
        cid = lax.axis_index("c")
        w = lax.axis_index("s")
        zeros16 = jnp.zeros((16,), jnp.float32)

        def zb(i, carry):
            for j in range(DW // 16):
                zbuf[i, pl.ds(j * 16, 16)] = zeros16
            return carry

        lax.fori_loop(0, 64, zb, 0)
        pltpu.sync_copy(src_hbm.at[pl.ds(w * STEPS, STEPS)], src_v)
        pltpu.sync_copy(dst_hbm.at[pl.ds(w * STEPS, STEPS)], dst_v)

        def run(y_ref, win_lo, win_hi, bases, c_outs):
            for acc in accs:
                for i in range(A_PT // 64):
                    pltpu.sync_copy(
                        zbuf, acc.at[pl.ds(w * A_PT + i * 64, 64)]
                    )
            plsc.subcore_barrier()

            def gather(s, rows_b, sem_b):
                pltpu.async_copy(y_ref.at[src_v.at[s]], rows_b, sem_b)

            def gwait(rows_b, sem_b):
                pltpu.make_async_copy(
                    y_ref.at[src_v.at[0]], rows_b, sem_b
                ).wait()

            def scat(s, rows_b, widx_b):
                for acc, base in zip(accs, bases):
                    for j in range(K // 16):
                        d16 = dst_v[s, pl.ds(j * 16, 16)]
                        local = d16 - base
                        ok = jnp.logical_and(local >= 0, local < N_HALF)
                        widx_b[pl.ds(j * 16, 16)] = jnp.where(
                            ok, local, N_HALF + (d16 & 63)
                        )
                    pltpu.sync_copy(rows_b, acc.at[widx_b], add=True)

            gather(win_lo, rows0, gsem0)
            n_pairs = (win_hi - win_lo) // 2

            def pair(p, carry):
                s0 = win_lo + 2 * p
                gather(s0 + 1, rows1, gsem1)
                gwait(rows0, gsem0)
                scat(s0, rows0, widx0)

                @pl.when(p < n_pairs - 1)
                def _():
                    gather(s0 + 2, rows0, gsem0)

                gwait(rows1, gsem1)
                scat(s0 + 1, rows1, widx1)
                return carry

            lax.fori_loop(0, n_pairs, pair, 0)
            plsc.subcore_barrier()
            for acc, out_ref in zip(accs, c_outs):
                pltpu.sync_copy(
                    acc.at[pl.ds(w * A_PT, A_PT)],
                    out_ref.at[pl.ds(w * A_PT, A_PT)],
                )

        o = 0
        out_base = []
        for cu in units:
            out_base.append(o)
            o += n_accs * len(cu)
        for c in range(NC):
            @pl.when(cid == c)
            def _(c=c):
                for i, (t, lo, hi, bases) in enumerate(units[c]):
                    ob = out_base[c] + i * n_accs
                    run(ys[t], lo, hi, bases, outs[ob: ob + n_accs])

    return scatter_kernel


_scatter_hid = _make_scatter_kernel(
    1,
    [[(0, 0, STEPS, [0]), (0, 0, STEPS, [N_HALF])],
     [(1, 0, STEPS, [0]), (1, 0, STEPS, [N_HALF])]],
)
_scatter_out = _make_scatter_kernel(
    1,
    [[(0, 0, STEPS, [0])], [(1, 0, STEPS, [N_HALF])]],
)


def _mm1_body(x_ref, w1_ref, deg_ref, ya_ref, yb_ref):
    dinv = lax.rsqrt(deg_ref[...] + 1.0)
    z = jnp.dot(x_ref[...], w1_ref[...], preferred_element_type=jnp.float32)
    y = z * dinv
    ya_ref[...] = y[:, :DW]
    yb_ref[...] = y[:, DW:]


def _mm2_body(sa_ref, sb_ref, ya_ref, yb_ref, deg_ref, b1_ref, w2_ref,
              y2a_ref, y2b_ref):
    dinv = lax.rsqrt(deg_ref[...] + 1.0)
    h = jnp.concatenate(
        [sa_ref[...] + ya_ref[...], sb_ref[...] + yb_ref[...]], axis=1
    )
    h = jnp.maximum(h * dinv + b1_ref[...], 0.0)
    y2 = jnp.dot(h, w2_ref[...], preferred_element_type=jnp.float32) * dinv
    y2a_ref[...] = y2
    y2b_ref[...] = y2


def _final_body(s_ref, y_ref, deg_ref, b2_ref, o_ref):
    dinv = lax.rsqrt(deg_ref[...] + 1.0)
    o_ref[...] = (s_ref[...] + y_ref[...]) * dinv + b2_ref[...]


def _row_spec(d):
    return pl.BlockSpec((BLK, d), lambda i: (i, 0))


def _full_spec(r, c):
    return pl.BlockSpec((r, c), lambda i: (0, 0))


def kernel(x, edge_index, W1, b1, W2, b2):
    src = edge_index[0]
    dst = edge_index[1]
    pad_src = jnp.zeros((NS, PAD), jnp.int32)
    pad_dst = jnp.broadcast_to(
        N + (jnp.arange(PAD, dtype=jnp.int32) % (N_ACC - N)), (NS, PAD)
    )
    src_pad = jnp.concatenate([src.reshape(NS, E_T), pad_src], axis=1)
    src_pad = src_pad.reshape(NS * STEPS, K)
    dst_pad = jnp.concatenate([dst.reshape(NS, E_T), pad_dst], axis=1)
    dst_pad = dst_pad.reshape(NS * STEPS, K)

    deg = _deg_kernel(dst_pad)
    deg_col = deg[:N].reshape(N, 1)

    y1a, y1b = pl.pallas_call(
        _mm1_body,
        grid=(NB,),
        in_specs=[
            _row_spec(D_IN),
            _full_spec(D_IN, D_HID),
            _row_spec(1),
        ],
        out_specs=(_row_spec(DW), _row_spec(DW)),
        out_shape=(
            jax.ShapeDtypeStruct((N, DW), jnp.float32),
            jax.ShapeDtypeStruct((N, DW), jnp.float32),
        ),
    )(x, W1, deg_col)

    u00, u01, u10, u11 = _scatter_hid(y1a, y1b, src_pad, dst_pad)
    s1a = jnp.concatenate([u00[:N_HALF], u01[:N_HALF]], axis=0)
    s1b = jnp.concatenate([u10[:N_HALF], u11[:N_HALF]], axis=0)

    y2 = pl.pallas_call(
        _mm2_body,
        grid=(NB,),
        in_specs=[
            _row_spec(DW),
            _row_spec(DW),
            _row_spec(DW),
            _row_spec(DW),
            _row_spec(1),
            _full_spec(1, D_HID),
            _full_spec(D_HID, D_OUT),
        ],
        out_specs=(_row_spec(D_OUT), _row_spec(D_OUT)),
        out_shape=(
            jax.ShapeDtypeStruct((N, D_OUT), jnp.float32),
            jax.ShapeDtypeStruct((N, D_OUT), jnp.float32),
        ),
    )(s1a, s1b, y1a, y1b, deg_col, b1.reshape(1, D_HID), W2)
    y2, y2_copy = y2

    v0, v1 = _scatter_out(y2, y2_copy, src_pad, dst_pad)
    s2 = jnp.concatenate([v0[:N_HALF], v1[:N_HALF]], axis=0)

    out = pl.pallas_call(
        _final_body,
        grid=(NB,),
        in_specs=[
            _row_spec(D_OUT),
            _row_spec(D_OUT),
            _row_spec(1),
            _full_spec(1, D_OUT),
        ],
        out_specs=_row_spec(D_OUT),
        out_shape=jax.ShapeDtypeStruct((N, D_OUT), jnp.float32),
    )(s2, y2, deg_col, b2.reshape(1, D_OUT))

    return out


# speedup vs baseline: 1.2862x; 1.0008x over previous
import functools

import jax
import jax.numpy as jnp
from jax import lax
from jax.experimental import pallas as pl
from jax.experimental.pallas import tpu as pltpu
from jax.experimental.pallas import tpu_sc as plsc

N = 10000
E = 320000
D_IN = 128
D_HID = 256
D_OUT = 128

NC = 2
NS = 16
K = 128
E_T = E // NS
STEPS = 160
PAD = STEPS * K - E_T
N_ACC = 10240
ROWS_PER_TILE = N_ACC // NS
NB = 10
BLK = N // NB

_mesh = plsc.VectorSubcoreMesh(
    core_axis_name="c", subcore_axis_name="s", num_cores=NC, num_subcores=NS
)


@functools.partial(
    pl.kernel,
    out_type=jax.ShapeDtypeStruct((N_ACC,), jnp.float32),
    mesh=_mesh,
    scratch_types=[
        pltpu.VMEM((STEPS, K), jnp.int32),
        pltpu.VMEM((K,), jnp.float32),
        pltpu.VMEM((ROWS_PER_TILE,), jnp.float32),
        pltpu.VMEM_SHARED((N_ACC,), jnp.float32),
    ],
)
def _deg_kernel(dst_hbm, deg_out, dst_v, ones_v, zbuf, deg_sp):
    cid = lax.axis_index("c")
    w = lax.axis_index("s")
    zeros16 = jnp.zeros((16,), jnp.float32)
    ones16 = jnp.ones((16,), jnp.float32)

    def zb(i, carry):
        zbuf[pl.ds(i * 16, 16)] = zeros16
        return carry

    lax.fori_loop(0, ROWS_PER_TILE // 16, zb, 0)
    for j in range(K // 16):
        ones_v[pl.ds(j * 16, 16)] = ones16
    pltpu.sync_copy(zbuf, deg_sp.at[pl.ds(w * ROWS_PER_TILE, ROWS_PER_TILE)])
    pltpu.sync_copy(dst_hbm.at[pl.ds(w * STEPS, STEPS)], dst_v)
    plsc.subcore_barrier()

    def step(s, carry):
        pltpu.sync_copy(ones_v, deg_sp.at[dst_v.at[s]], add=True)
        return carry

    lax.fori_loop(0, STEPS, step, 0)
    plsc.subcore_barrier()

    @pl.when(jnp.logical_and(cid == 0, w == 0))
    def _():
        pltpu.sync_copy(deg_sp, deg_out)


DW = 128
N_HALF = 5000
A_ROWS = 5120
A_PT = A_ROWS // NS


def _make_scatter_kernel(n_accs, units):
    n_tables = max(u[0] for cu in units for u in cu) + 1
    n_out = n_accs * sum(len(cu) for cu in units)

    @functools.partial(
        pl.kernel,
        out_type=tuple(
            jax.ShapeDtypeStruct((A_ROWS, DW), jnp.float32)
            for _ in range(n_out)
        ),
        mesh=_mesh,
        scratch_types=(
            [
                pltpu.VMEM((STEPS, K), jnp.int32),
                pltpu.VMEM((STEPS, K), jnp.int32),
                pltpu.VMEM((K, DW), jnp.float32),
                pltpu.VMEM((K, DW), jnp.float32),
                pltpu.VMEM((64, DW), jnp.float32),
            ]
            + [pltpu.VMEM_SHARED((A_ROWS, DW), jnp.float32)] * n_accs
            + [pltpu.SemaphoreType.DMA, pltpu.SemaphoreType.DMA]
        ),
    )
    def scatter_kernel(*refs):
        ys = refs[:n_tables]
        src_hbm, dst_hbm = refs[n_tables], refs[n_tables + 1]
        outs = refs[n_tables + 2: n_tables + 2 + n_out]
        sc = refs[n_tables + 2 + n_out:]
        src_v, dst_v, rows0, rows1, zbuf = sc[:5]
        accs = sc[5: 5 + n_accs]
        gsem0, gsem1 = sc[5 + n_accs:]
        cid = lax.axis_index("c")
        w = lax.axis_index("s")
        zeros16 = jnp.zeros((16,), jnp.float32)

        def zb(i, carry):
            for j in range(DW // 16):
                zbuf[i, pl.ds(j * 16, 16)] = zeros16
            return carry

        lax.fori_loop(0, 64, zb, 0)
        pltpu.sync_copy(src_hbm.at[pl.ds(w * STEPS, STEPS)], src_v)

        def run(y_ref, win_lo, win_hi, bases, c_outs):
            acc0 = accs[0]
            base = bases[0]
            for a in accs:
                for i in range(A_PT // 64):
                    pltpu.sync_copy(
                        zbuf, a.at[pl.ds(w * A_PT + i * 64, 64)]
                    )
            pltpu.sync_copy(dst_hbm.at[pl.ds(w * STEPS, STEPS)], dst_v)

            def mk(s, carry):
                for j in range(K // 16):
                    d16 = dst_v[s, pl.ds(j * 16, 16)]
                    local = d16 - base
                    ok = jnp.logical_and(local >= 0, local < N_HALF)
                    dst_v[s, pl.ds(j * 16, 16)] = jnp.where(
                        ok, local, N_HALF + (d16 & 63)
                    )
                return carry

            lax.fori_loop(win_lo, win_hi, mk, 0)
            plsc.subcore_barrier()

            def gather(s, rows_b, sem_b):
                pltpu.async_copy(y_ref.at[src_v.at[s]], rows_b, sem_b)

            def gwait(rows_b, sem_b):
                pltpu.make_async_copy(
                    y_ref.at[src_v.at[0]], rows_b, sem_b
                ).wait()

            def scat(s, rows_b):
                pltpu.sync_copy(rows_b, acc0.at[dst_v.at[s]], add=True)

            gather(win_lo, rows0, gsem0)
            n_pairs = (win_hi - win_lo) // 2

            def pair(p, carry):
                s0 = win_lo + 2 * p
                gather(s0 + 1, rows1, gsem1)
                gwait(rows0, gsem0)
                scat(s0, rows0)

                @pl.when(p < n_pairs - 1)
                def _():
                    gather(s0 + 2, rows0, gsem0)

                gwait(rows1, gsem1)
                scat(s0 + 1, rows1)
                return carry

            lax.fori_loop(0, n_pairs, pair, 0)
            plsc.subcore_barrier()
            for acc, out_ref in zip(accs, c_outs):
                pltpu.sync_copy(
                    acc.at[pl.ds(w * A_PT, A_PT)],
                    out_ref.at[pl.ds(w * A_PT, A_PT)],
                )

        o = 0
        out_base = []
        for cu in units:
            out_base.append(o)
            o += n_accs * len(cu)
        for c in range(NC):
            @pl.when(cid == c)
            def _(c=c):
                for i, (t, lo, hi, bases) in enumerate(units[c]):
                    ob = out_base[c] + i * n_accs
                    run(ys[t], lo, hi, bases, outs[ob: ob + n_accs])

    return scatter_kernel


_scatter_hid = _make_scatter_kernel(
    1,
    [[(0, 0, STEPS, [0]), (0, 0, STEPS, [N_HALF])],
     [(1, 0, STEPS, [0]), (1, 0, STEPS, [N_HALF])]],
)
_scatter_out = _make_scatter_kernel(
    1,
    [[(0, 0, STEPS, [0])], [(1, 0, STEPS, [N_HALF])]],
)


def _mm1_body(x_ref, w1_ref, deg_ref, ya_ref, yb_ref):
    dinv = lax.rsqrt(deg_ref[...] + 1.0)
    z = jnp.dot(x_ref[...], w1_ref[...], preferred_element_type=jnp.float32)
    y = z * dinv
    ya_ref[...] = y[:, :DW]
    yb_ref[...] = y[:, DW:]


def _mm2_body(sa_ref, sb_ref, ya_ref, yb_ref, deg_ref, b1_ref, w2_ref,
              y2a_ref, y2b_ref):
    dinv = lax.rsqrt(deg_ref[...] + 1.0)
    h = jnp.concatenate(
        [sa_ref[...] + ya_ref[...], sb_ref[...] + yb_ref[...]], axis=1
    )
    h = jnp.maximum(h * dinv + b1_ref[...], 0.0)
    y2 = jnp.dot(h, w2_ref[...], preferred_element_type=jnp.float32) * dinv
    y2a_ref[...] = y2
    y2b_ref[...] = y2


def _final_body(s_ref, y_ref, deg_ref, b2_ref, o_ref):
    dinv = lax.rsqrt(deg_ref[...] + 1.0)
    o_ref[...] = (s_ref[...] + y_ref[...]) * dinv + b2_ref[...]


def _row_spec(d):
    return pl.BlockSpec((BLK, d), lambda i: (i, 0))


def _full_spec(r, c):
    return pl.BlockSpec((r, c), lambda i: (0, 0))


def kernel(x, edge_index, W1, b1, W2, b2):
    src = edge_index[0]
    dst = edge_index[1]
    pad_src = jnp.zeros((NS, PAD), jnp.int32)
    pad_dst = jnp.broadcast_to(
        N + (jnp.arange(PAD, dtype=jnp.int32) % (N_ACC - N)), (NS, PAD)
    )
    src_pad = jnp.concatenate([src.reshape(NS, E_T), pad_src], axis=1)
    src_pad = src_pad.reshape(NS * STEPS, K)
    dst_pad = jnp.concatenate([dst.reshape(NS, E_T), pad_dst], axis=1)
    dst_pad = dst_pad.reshape(NS * STEPS, K)

    deg = _deg_kernel(dst_pad)
    deg_col = deg[:N].reshape(N, 1)

    y1a, y1b = pl.pallas_call(
        _mm1_body,
        grid=(NB,),
        in_specs=[
            _row_spec(D_IN),
            _full_spec(D_IN, D_HID),
            _row_spec(1),
        ],
        out_specs=(_row_spec(DW), _row_spec(DW)),
        out_shape=(
            jax.ShapeDtypeStruct((N, DW), jnp.float32),
            jax.ShapeDtypeStruct((N, DW), jnp.float32),
        ),
    )(x, W1, deg_col)

    u00, u01, u10, u11 = _scatter_hid(y1a, y1b, src_pad, dst_pad)
    s1a = jnp.concatenate([u00[:N_HALF], u01[:N_HALF]], axis=0)
    s1b = jnp.concatenate([u10[:N_HALF], u11[:N_HALF]], axis=0)

    y2 = pl.pallas_call(
        _mm2_body,
        grid=(NB,),
        in_specs=[
            _row_spec(DW),
            _row_spec(DW),
            _row_spec(DW),
            _row_spec(DW),
            _row_spec(1),
            _full_spec(1, D_HID),
            _full_spec(D_HID, D_OUT),
        ],
        out_specs=(_row_spec(D_OUT), _row_spec(D_OUT)),
        out_shape=(
            jax.ShapeDtypeStruct((N, D_OUT), jnp.float32),
            jax.ShapeDtypeStruct((N, D_OUT), jnp.float32),
        ),
    )(s1a, s1b, y1a, y1b, deg_col, b1.reshape(1, D_HID), W2)
    y2, y2_copy = y2

    v0, v1 = _scatter_out(y2, y2_copy, src_pad, dst_pad)
    s2 = jnp.concatenate([v0[:N_HALF], v1[:N_HALF]], axis=0)

    out = pl.pallas_call(
        _final_body,
        grid=(NB,),
        in_specs=[
            _row_spec(D_OUT),
            _row_spec(D_OUT),
            _row_spec(1),
            _full_spec(1, D_OUT),
        ],
        out_specs=_row_spec(D_OUT),
        out_shape=jax.ShapeDtypeStruct((N, D_OUT), jnp.float32),
    )(s2, y2, deg_col, b2.reshape(1, D_OUT))

    return out
